# Initial kernel scaffold; baseline (speedup 1.0000x reference)
import jax
import jax.numpy as jnp
from jax.experimental import pallas as pl


def kernel(x_user, x_restaurant, edge_label_index):
    raise NotImplementedError("write your pallas kernel here")


# speedup vs baseline: 5.2005x; 5.2005x over previous
import functools

import jax
import jax.numpy as jnp
from jax import lax
from jax.experimental import pallas as pl
from jax.experimental.pallas import tpu as pltpu
from jax.experimental.pallas import tpu_sc as plsc

E = 320000
V = 10000
D = 128
NC = 2
NS = 16
L = 16
NW = NC * NS
EW = E // NW
CH = 400
NCHUNK = EW // CH
NSTREAM = 5
SR = CH // NSTREAM
NG = CH // L


def _body(xu, xr, iu, ir, out, idxu_v, idxr_v, urows, rrows, pbuf, out_v, sem):
    c = lax.axis_index("c")
    s = lax.axis_index("s")
    wid = s * NC + c
    base_w = wid * EW

    pltpu.sync_copy(iu.at[pl.ds(base_w, EW)], idxu_v)
    pltpu.sync_copy(ir.at[pl.ds(base_w, EW)], idxr_v)

    def chunk_body(ci, carry):
        cbase = ci * CH
        cps = []
        for t in range(NSTREAM):
            off = cbase + t * SR
            cps.append(pltpu.async_copy(
                xu.at[idxu_v.at[pl.ds(off, SR)]],
                urows.at[pl.ds(t * SR, SR)], sem))
            cps.append(pltpu.async_copy(
                xr.at[idxr_v.at[pl.ds(off, SR)]],
                rrows.at[pl.ds(t * SR, SR)], sem))
        for cp in cps:
            cp.wait()

        def group_body(g, gcarry):
            for i in range(L):
                e = g * L + i
                p = None
                for k in range(D // L):
                    t = urows[e, pl.ds(k * L, L)] * rrows[e, pl.ds(k * L, L)]
                    p = t if p is None else p + t
                pbuf[pl.ds(i * L, L)] = p
            ebase = lax.iota(jnp.int32, L) * L
            acc = plsc.load_gather(pbuf, [ebase])
            for j in range(1, L):
                acc = acc + plsc.load_gather(pbuf, [ebase + j])
            out_v[pl.ds(g * L, L)] = acc
            return gcarry

        lax.fori_loop(0, NG, group_body, 0, unroll=False)
        pltpu.sync_copy(out_v, out.at[pl.ds(base_w + cbase, CH)])
        return carry

    lax.fori_loop(0, NCHUNK, chunk_body, 0, unroll=False)


@jax.jit
def _run(xu, xr, iu, ir):
    mesh = plsc.VectorSubcoreMesh(
        core_axis_name="c", subcore_axis_name="s", num_cores=NC,
        num_subcores=NS)
    return pl.kernel(
        _body,
        out_type=jax.ShapeDtypeStruct((E,), jnp.float32),
        mesh=mesh,
        compiler_params=pltpu.CompilerParams(needs_layout_passes=False),
        scratch_types=[
            pltpu.VMEM((EW,), jnp.int32),
            pltpu.VMEM((EW,), jnp.int32),
            pltpu.VMEM((CH, D), jnp.float32),
            pltpu.VMEM((CH, D), jnp.float32),
            pltpu.VMEM((L * L,), jnp.float32),
            pltpu.VMEM((CH,), jnp.float32),
            pltpu.SemaphoreType.DMA,
        ],
    )(xu, xr, iu, ir)


def kernel(x_user, x_restaurant, edge_label_index):
    eli = edge_label_index.astype(jnp.int32)
    return _run(x_user, x_restaurant, eli[0], eli[1])


# speedup vs baseline: 8.1440x; 1.5660x over previous
import jax
import jax.numpy as jnp
from jax import lax
from jax.experimental import pallas as pl
from jax.experimental.pallas import tpu as pltpu
from jax.experimental.pallas import tpu_sc as plsc

E = 320000
V = 10000
D = 128
NC = 2
NS = 16
L = 16
NW = NC * NS
EW = E // NW
CHB = 80
NCHUNK = EW // CHB
NBUF = 4
NGB = CHB // L


def _body(xu, xr, iu, ir, out, idxu_v, idxr_v, urows, rrows, pbuf,
          out_all, sem):
    c = lax.axis_index("c")
    s = lax.axis_index("s")
    wid = s * NC + c
    base_w = wid * EW

    pltpu.sync_copy(iu.at[pl.ds(base_w, EW)], idxu_v)
    pltpu.sync_copy(ir.at[pl.ds(base_w, EW)], idxr_v)

    def issue(ci, b):
        pltpu.async_copy(
            xu.at[idxu_v.at[pl.ds(ci * CHB, CHB)]], urows.at[b], sem.at[b])
        pltpu.async_copy(
            xr.at[idxr_v.at[pl.ds(ci * CHB, CHB)]], rrows.at[b], sem.at[b])

    for b in range(NBUF):
        issue(b, b)

    def chunk_body(ci, carry):
        b = lax.rem(ci, NBUF)
        pltpu.make_async_copy(xu.at[pl.ds(0, CHB)], urows.at[b],
                              sem.at[b]).wait()
        pltpu.make_async_copy(xu.at[pl.ds(0, CHB)], rrows.at[b],
                              sem.at[b]).wait()

        def group_body(g, gcarry):
            for i in range(L):
                e = g * L + i
                p = None
                for k in range(D // L):
                    t = urows[b, e, pl.ds(k * L, L)] * rrows[b, e, pl.ds(k * L, L)]
                    p = t if p is None else p + t
                pbuf[pl.ds(i * L, L)] = p
            ebase = lax.iota(jnp.int32, L) * L
            acc = plsc.load_gather(pbuf, [ebase])
            for j in range(1, L):
                acc = acc + plsc.load_gather(pbuf, [ebase + j])
            out_all[pl.ds(ci * CHB + g * L, L)] = acc
            return gcarry

        lax.fori_loop(0, NGB, group_body, 0, unroll=False)

        @pl.when(ci < NCHUNK - NBUF)
        def _():
            issue(ci + NBUF, b)

        return carry

    lax.fori_loop(0, NCHUNK, chunk_body, 0, unroll=False)
    pltpu.sync_copy(out_all, out.at[pl.ds(base_w, EW)])


@jax.jit
def _run(xu, xr, iu, ir):
    mesh = plsc.VectorSubcoreMesh(
        core_axis_name="c", subcore_axis_name="s", num_cores=NC,
        num_subcores=NS)
    return pl.kernel(
        _body,
        out_type=jax.ShapeDtypeStruct((E,), jnp.float32),
        mesh=mesh,
        compiler_params=pltpu.CompilerParams(needs_layout_passes=False),
        scratch_types=[
            pltpu.VMEM((EW,), jnp.int32),
            pltpu.VMEM((EW,), jnp.int32),
            pltpu.VMEM((NBUF, CHB, D), jnp.float32),
            pltpu.VMEM((NBUF, CHB, D), jnp.float32),
            pltpu.VMEM((L * L,), jnp.float32),
            pltpu.VMEM((EW,), jnp.float32),
            pltpu.SemaphoreType.DMA((NBUF,)),
        ],
    )(xu, xr, iu, ir)


def kernel(x_user, x_restaurant, edge_label_index):
    eli = edge_label_index.astype(jnp.int32)
    return _run(x_user, x_restaurant, eli[0], eli[1])
